# baseline (speedup 1.0000x reference)
import functools

import jax
import jax.numpy as jnp
import numpy as np
from jax import lax
from jax.experimental import pallas as pl
from jax.experimental.pallas import tpu as pltpu
from jax.experimental.pallas import tpu_sc as plsc

_NUM_LEVELS = 16
_BASE_RES = 16
_LOG2 = 19
_V = 2 ** _LOG2
_FEAT = 2
_N = 524288
_MASK = _V - 1
_P2 = np.int32(np.uint32(2654435761).astype(np.int32))
_P3 = np.int32(805459861)

_LANES = 128
_R = _N // _LANES
_RB = 256
_NW = 32
_CHUNK = _N // _NW
_C = 8192


def _hash_body(x_ref, o_ref):
    x = x_ref[0]
    y = x_ref[1]
    z = x_ref[2]
    for lvl in range(_NUM_LEVELS):
        res = np.float32(_BASE_RES * (2 ** lvl))
        xi = (x * res).astype(jnp.int32)
        yi = (y * res).astype(jnp.int32)
        zi = (z * res).astype(jnp.int32)
        h = (xi ^ (yi * _P2) ^ (zi * _P3)) & _MASK
        o_ref[lvl] = h | (lvl << _LOG2)


def _hash_indices(xt):
    return pl.pallas_call(
        _hash_body,
        grid=(_R // _RB,),
        in_specs=[
            pl.BlockSpec(
                (3, _RB, _LANES),
                lambda i: (jnp.int32(0), i, jnp.int32(0)),
            )
        ],
        out_specs=pl.BlockSpec(
            (_NUM_LEVELS, _RB, _LANES),
            lambda i: (jnp.int32(0), i, jnp.int32(0)),
        ),
        out_shape=jax.ShapeDtypeStruct((_NUM_LEVELS, _R, _LANES), jnp.int32),
    )(xt)


def _sc_gather_body(idx_hbm, table_hbm, out_hbm, idx_v, rows_v, sem):
    wid = lax.axis_index("s") * 2 + lax.axis_index("c")
    base0 = wid * jnp.int32(_CHUNK)
    for lvl in range(_NUM_LEVELS):
        lv = jnp.int32(lvl)
        for j in range(_CHUNK // _C):
            base = base0 + jnp.int32(j * _C)
            pltpu.sync_copy(idx_hbm.at[lv, pl.ds(base, _C)], idx_v)
            pltpu.async_copy(table_hbm.at[idx_v], rows_v, sem).wait()
            pltpu.sync_copy(rows_v, out_hbm.at[lv, pl.ds(base, _C)])


@functools.cache
def _sc_gather():
    return functools.partial(
        pl.kernel,
        mesh=plsc.VectorSubcoreMesh(core_axis_name="c", subcore_axis_name="s"),
        compiler_params=pltpu.CompilerParams(use_tc_tiling_on_sc=False),
        out_type=jax.ShapeDtypeStruct((_NUM_LEVELS, _N, _FEAT), jnp.float32),
        scratch_types=[
            pltpu.VMEM((_C,), jnp.int32),
            pltpu.VMEM((_C, _FEAT), jnp.float32),
            pltpu.SemaphoreType.DMA,
        ],
    )(_sc_gather_body)


def kernel(xyz, grids):
    xt = xyz.T.reshape(3, _R, _LANES)
    idx = _hash_indices(xt).reshape(_NUM_LEVELS, _N)
    table = grids.reshape(_NUM_LEVELS * _V, _FEAT)
    out = _sc_gather()(idx, table)
    return out.transpose(1, 0, 2).reshape(_N, _NUM_LEVELS * _FEAT)


# speedup vs baseline: 1.3647x; 1.3647x over previous
import functools

import jax
import jax.numpy as jnp
import numpy as np
from jax import lax
from jax.experimental import pallas as pl
from jax.experimental.pallas import tpu as pltpu
from jax.experimental.pallas import tpu_sc as plsc

_NUM_LEVELS = 16
_BASE_RES = 16
_LOG2 = 19
_V = 2 ** _LOG2
_FEAT = 2
_N = 524288
_MASK = _V - 1
_P2 = np.uint32(2654435761).astype(np.int32)
_P3 = np.int32(805459861)

_NW = 32
_CHUNK = _N // _NW
_C = 512
_NCH = _N // _C
_G = _C // 16

_MESH = dict(core_axis_name="c", subcore_axis_name="s")


def _worker_id():
    return lax.axis_index("s") * 2 + lax.axis_index("c")


def _hash_body(xyz_hbm, idx_hbm, xyz_v, idx_v, lane3):
    wid = _worker_id()
    chunk0 = wid * jnp.int32(_CHUNK // _C)
    for j in range(_CHUNK // _C):
        chunk = chunk0 + jnp.int32(j)
        pltpu.sync_copy(xyz_hbm.at[pl.ds(chunk * jnp.int32(3 * _C), 3 * _C)], xyz_v)

        def hash_group(g, gi):
            off = gi * jnp.int32(48)
            xg = plsc.load_gather(xyz_v, [lane3 + off])
            yg = plsc.load_gather(xyz_v, [lane3 + (off + jnp.int32(1))])
            zg = plsc.load_gather(xyz_v, [lane3 + (off + jnp.int32(2))])
            s = pl.ds(gi * 16, 16)
            for lvl in range(_NUM_LEVELS):
                res = jnp.float32(_BASE_RES * (2 ** lvl))
                xi = (xg * res).astype(jnp.int32)
                yi = (yg * res).astype(jnp.int32)
                zi = (zg * res).astype(jnp.int32)
                h = (xi ^ (yi * _P2) ^ (zi * _P3)) & jnp.int32(_MASK)
                idx_v[jnp.int32(lvl), s] = h + jnp.int32(lvl * _V)
            return gi + jnp.int32(1)

        lax.fori_loop(0, _G, hash_group, jnp.int32(0))
        pltpu.sync_copy(idx_v, idx_hbm.at[chunk])


def _hash_entry(xyz_v_ref, idx_hbm, xyz_v, idx_v):
    lane3 = lax.iota(jnp.int32, 16) * jnp.int32(3)
    _hash_body(xyz_v_ref, idx_hbm, xyz_v, idx_v, lane3)


@functools.cache
def _hash_kernel():
    return functools.partial(
        pl.kernel,
        mesh=plsc.VectorSubcoreMesh(**_MESH),
        compiler_params=pltpu.CompilerParams(
            use_tc_tiling_on_sc=False, needs_layout_passes=False
        ),
        out_type=jax.ShapeDtypeStruct((_NCH, _NUM_LEVELS, _C), jnp.int32),
        scratch_types=[
            pltpu.VMEM((3 * _C,), jnp.float32),
            pltpu.VMEM((_NUM_LEVELS, _C), jnp.int32),
        ],
    )(_hash_entry)


def _gather_body(idx_hbm, table_hbm, tmp_hbm, idx_v, rows_v, sem):
    wid = _worker_id()
    chunk0 = wid * jnp.int32(_CHUNK // _C)
    for j in range(_CHUNK // _C):
        chunk = chunk0 + jnp.int32(j)
        pltpu.sync_copy(idx_hbm.at[chunk], idx_v)
        copies = []
        for lvl in range(_NUM_LEVELS):
            copies.append(
                pltpu.async_copy(
                    table_hbm.at[idx_v.at[jnp.int32(lvl)]],
                    rows_v.at[jnp.int32(lvl)],
                    sem,
                )
            )
        for cp in copies:
            cp.wait()
        pltpu.sync_copy(rows_v, tmp_hbm.at[chunk])


@functools.cache
def _gather_kernel():
    return functools.partial(
        pl.kernel,
        mesh=plsc.VectorSubcoreMesh(**_MESH),
        compiler_params=pltpu.CompilerParams(use_tc_tiling_on_sc=False),
        out_type=jax.ShapeDtypeStruct((_NCH, _NUM_LEVELS, _C, _FEAT), jnp.float32),
        scratch_types=[
            pltpu.VMEM((_NUM_LEVELS, _C), jnp.int32),
            pltpu.VMEM((_NUM_LEVELS, _C, _FEAT), jnp.float32),
            pltpu.SemaphoreType.DMA,
        ],
    )(_gather_body)


def _combine_body(tmp_hbm, out_hbm, rows_v, tile_v):
    wid = _worker_id()
    chunk0 = wid * jnp.int32(_CHUNK // _C)
    lanes = lax.iota(jnp.int32, 16)
    f_pat = lanes & jnp.int32(1)
    l_half = lanes >> 1
    for j in range(_CHUNK // _C):
        chunk = chunk0 + jnp.int32(j)
        pltpu.sync_copy(tmp_hbm.at[chunk], rows_v)

        def point(c, ci):
            c_pat = jnp.zeros((16,), jnp.int32) + ci
            for h in range(2):
                lvl_pat = l_half + jnp.int32(8 * h)
                seg = plsc.load_gather(rows_v, [lvl_pat, c_pat, f_pat])
                tile_v[ci, pl.ds(jnp.int32(16 * h), 16)] = seg
            return ci + jnp.int32(1)

        lax.fori_loop(0, _C, point, jnp.int32(0))
        pltpu.sync_copy(tile_v, out_hbm.at[pl.ds(chunk * jnp.int32(_C), _C)])


@functools.cache
def _combine_kernel():
    return functools.partial(
        pl.kernel,
        mesh=plsc.VectorSubcoreMesh(**_MESH),
        compiler_params=pltpu.CompilerParams(
            use_tc_tiling_on_sc=False, needs_layout_passes=False
        ),
        out_type=jax.ShapeDtypeStruct((_N, _NUM_LEVELS * _FEAT), jnp.float32),
        scratch_types=[
            pltpu.VMEM((_NUM_LEVELS, _C, _FEAT), jnp.float32),
            pltpu.VMEM((_C, _NUM_LEVELS * _FEAT), jnp.float32),
        ],
    )(_combine_body)


def kernel(xyz, grids):
    xyz_flat = xyz.reshape(_N * 3)
    table = grids.reshape(_NUM_LEVELS * _V, _FEAT)
    idx = _hash_kernel()(xyz_flat)
    tmp = _gather_kernel()(idx, table)
    return _combine_kernel()(tmp)


# speedup vs baseline: 1.6620x; 1.2179x over previous
import functools

import jax
import jax.numpy as jnp
import numpy as np
from jax import lax
from jax.experimental import pallas as pl
from jax.experimental.pallas import tpu as pltpu
from jax.experimental.pallas import tpu_sc as plsc

_NUM_LEVELS = 16
_BASE_RES = 16
_LOG2 = 19
_V = 2 ** _LOG2
_FEAT = 2
_N = 524288
_MASK = _V - 1
_P2 = np.uint32(2654435761).astype(np.int32)
_P3 = np.int32(805459861)

_NW = 32
_CHUNK = _N // _NW
_C = 512
_NCH = _N // _C
_G = _C // 16
_B = _C * _NUM_LEVELS

_MESH = dict(core_axis_name="c", subcore_axis_name="s")


def _worker_id():
    return lax.axis_index("s") * 2 + lax.axis_index("c")


def _hash_body(xyz_hbm, idx_hbm, xyz_v, idx_v):
    wid = _worker_id()
    chunk0 = wid * jnp.int32(_CHUNK // _C)
    lane3 = lax.iota(jnp.int32, 16) * jnp.int32(3)
    lane32 = lax.iota(jnp.int32, 16) * jnp.int32(32)
    for j in range(_CHUNK // _C):
        chunk = chunk0 + jnp.int32(j)
        pltpu.sync_copy(xyz_hbm.at[pl.ds(chunk * jnp.int32(3 * _C), 3 * _C)], xyz_v)

        def hash_group(g, gi):
            off = gi * jnp.int32(48)
            xg = plsc.load_gather(xyz_v, [lane3 + off])
            yg = plsc.load_gather(xyz_v, [lane3 + (off + jnp.int32(1))])
            zg = plsc.load_gather(xyz_v, [lane3 + (off + jnp.int32(2))])
            pos = lane32 + gi * jnp.int32(512)
            for lvl in range(_NUM_LEVELS):
                res = jnp.float32(_BASE_RES * (2 ** lvl))
                xi = (xg * res).astype(jnp.int32)
                yi = (yg * res).astype(jnp.int32)
                zi = (zg * res).astype(jnp.int32)
                h = (xi ^ (yi * _P2) ^ (zi * _P3)) & jnp.int32(_MASK)
                e0 = (h << jnp.int32(1)) + jnp.int32(2 * lvl * _V)
                p0 = pos + jnp.int32(2 * lvl)
                plsc.store_scatter(idx_v, [p0], e0)
                plsc.store_scatter(
                    idx_v, [p0 + jnp.int32(1)], e0 + jnp.int32(1)
                )
            return gi + jnp.int32(1)

        lax.fori_loop(0, _G, hash_group, jnp.int32(0))
        pltpu.sync_copy(idx_v, idx_hbm.at[chunk])


@functools.cache
def _hash_kernel():
    return functools.partial(
        pl.kernel,
        mesh=plsc.VectorSubcoreMesh(**_MESH),
        compiler_params=pltpu.CompilerParams(
            use_tc_tiling_on_sc=False, needs_layout_passes=False
        ),
        out_type=jax.ShapeDtypeStruct((_NCH, 2 * _B), jnp.int32),
        scratch_types=[
            pltpu.VMEM((3 * _C,), jnp.float32),
            pltpu.VMEM((2 * _B,), jnp.int32),
        ],
    )(_hash_body)


def _gather_body(idx_hbm, table_hbm, out_hbm, idx_v, rows_v, sem):
    wid = _worker_id()
    chunk0 = wid * jnp.int32(_CHUNK // _C)
    for j in range(_CHUNK // _C):
        chunk = chunk0 + jnp.int32(j)
        pltpu.sync_copy(idx_hbm.at[chunk], idx_v)
        pltpu.async_copy(table_hbm.at[idx_v], rows_v, sem).wait()
        pltpu.sync_copy(rows_v, out_hbm.at[pl.ds(chunk * jnp.int32(2 * _B), 2 * _B)])


@functools.cache
def _gather_kernel():
    return functools.partial(
        pl.kernel,
        mesh=plsc.VectorSubcoreMesh(**_MESH),
        compiler_params=pltpu.CompilerParams(use_tc_tiling_on_sc=False),
        out_type=jax.ShapeDtypeStruct((_N * _NUM_LEVELS * _FEAT,), jnp.float32),
        scratch_types=[
            pltpu.VMEM((2 * _B,), jnp.int32),
            pltpu.VMEM((2 * _B,), jnp.float32),
            pltpu.SemaphoreType.DMA,
        ],
    )(_gather_body)


def kernel(xyz, grids):
    xyz_flat = xyz.reshape(_N * 3)
    table = grids.reshape(_NUM_LEVELS * _V * _FEAT)
    idx = _hash_kernel()(xyz_flat)
    out = _gather_kernel()(idx, table)
    return out.reshape(_N, _NUM_LEVELS * _FEAT)


# speedup vs baseline: 26.8157x; 16.1345x over previous
import functools

import jax
import jax.numpy as jnp
import numpy as np
from jax import lax
from jax.experimental import pallas as pl
from jax.experimental.pallas import tpu as pltpu
from jax.experimental.pallas import tpu_sc as plsc

_NUM_LEVELS = 16
_BASE_RES = 16
_LOG2 = 19
_V = 2 ** _LOG2
_FEAT = 2
_N = 524288
_MASK = _V - 1
_P2 = np.uint32(2654435761).astype(np.int32)
_P3 = np.int32(805459861)

_NW = 32
_CHUNK = _N // _NW
_C = 1024
_G = _C // 16
_E = _C * 32
_SEG = _E // 4
_NBLK = _N // 128

_MESH = dict(core_axis_name="c", subcore_axis_name="s")


def _sc_body(xt_hbm, table_hbm, out_hbm, x_v, y_v, z_v, idx_v, rows_v, sem):
    wid = lax.axis_index("s") * 2 + lax.axis_index("c")
    chunk0 = wid * jnp.int32(_CHUNK // _C)

    for j in range(_CHUNK // _C):
        chunk = chunk0 + jnp.int32(j)
        base = chunk * jnp.int32(_C)
        pltpu.sync_copy(xt_hbm.at[jnp.int32(0), pl.ds(base, _C)], x_v)
        pltpu.sync_copy(xt_hbm.at[jnp.int32(1), pl.ds(base, _C)], y_v)
        pltpu.sync_copy(xt_hbm.at[jnp.int32(2), pl.ds(base, _C)], z_v)

        def hash_group(g, gi):
            s = pl.ds(gi * 16, 16)
            xg = x_v[s]
            yg = y_v[s]
            zg = z_v[s]
            sg = (gi >> 3) * jnp.int32(1024) + (gi & jnp.int32(7)) * jnp.int32(16)
            for lvl in range(_NUM_LEVELS):
                res = jnp.float32(_BASE_RES * (2 ** lvl))
                xi = (xg * res).astype(jnp.int32)
                yi = (yg * res).astype(jnp.int32)
                zi = (zg * res).astype(jnp.int32)
                h = (xi ^ (yi * _P2) ^ (zi * _P3)) & jnp.int32(_MASK)
                e0 = h + (h & jnp.int32(_MASK & ~127)) + jnp.int32(lvl << 20)
                j0 = 2 * lvl
                p0 = (j0 >> 3) * _SEG + (j0 & 7) * 128
                idx_v[pl.ds(sg + jnp.int32(p0), 16)] = e0
                j1 = j0 + 1
                p1 = (j1 >> 3) * _SEG + (j1 & 7) * 128
                idx_v[pl.ds(sg + jnp.int32(p1), 16)] = e0 + jnp.int32(128)
            return gi + jnp.int32(1)

        lax.fori_loop(0, _G, hash_group, jnp.int32(0))

        pltpu.async_copy(table_hbm.at[idx_v], rows_v, sem).wait()

        for jb in range(4):
            pltpu.sync_copy(
                rows_v.at[pl.ds(jnp.int32(jb * _SEG), _SEG)],
                out_hbm.at[
                    pl.ds(
                        jnp.int32(jb * _NBLK * 1024) + chunk * jnp.int32(_SEG),
                        _SEG,
                    )
                ],
            )


@functools.cache
def _sc_kernel():
    return functools.partial(
        pl.kernel,
        mesh=plsc.VectorSubcoreMesh(**_MESH),
        compiler_params=pltpu.CompilerParams(use_tc_tiling_on_sc=False),
        out_type=jax.ShapeDtypeStruct((_N * 32,), jnp.float32),
        scratch_types=[
            pltpu.VMEM((_C,), jnp.float32),
            pltpu.VMEM((_C,), jnp.float32),
            pltpu.VMEM((_C,), jnp.float32),
            pltpu.VMEM((_E,), jnp.int32),
            pltpu.VMEM((_E,), jnp.float32),
            pltpu.SemaphoreType.DMA,
        ],
    )(_sc_body)


def kernel(xyz, grids):
    table = (
        grids.reshape(_NUM_LEVELS, _V // 128, 128, _FEAT)
        .transpose(0, 1, 3, 2)
        .reshape(_NUM_LEVELS * _V * _FEAT)
    )
    xt = xyz.T
    out_p = _sc_kernel()(xt, table)
    return (
        out_p.reshape(4, _NBLK, 8, 128)
        .transpose(1, 3, 0, 2)
        .reshape(_N, _NUM_LEVELS * _FEAT)
    )
